# baseline (speedup 1.0000x reference)
import functools

import jax
import jax.numpy as jnp
from jax.experimental import pallas as pl
from jax.experimental.pallas import tpu as pltpu
from jax.experimental.pallas import tpu_sc as plsc

_H = 1024
_M = 1024
_K = 128
_B = 4
_T = 8192
_V = 64

_BT = 4096
_NB = (_B * _T) // _BT
_NW = 32
_RPW = (_B * _K) // _NW

_HIGH = jax.lax.Precision.HIGHEST


def _gate_body(x_ref, wg_ref, bg_ref, gate_ref, acc_ref):
    i = pl.program_id(0)
    x = x_ref[...]
    w = wg_ref[...]
    y = jnp.sum(x * w, axis=1)
    sig = jax.nn.sigmoid(y + bg_ref[0, 0])
    gate_ref[...] = sig.reshape(1, 1, _BT)

    @pl.when(i == 0)
    def _():
        acc_ref[...] = jnp.zeros_like(acc_ref)

    acc_ref[...] += jnp.sum(sig).reshape(1, 1)


def _gate_call(enc_flat, Wg, bg):
    return pl.pallas_call(
        _gate_body,
        grid=(_NB,),
        in_specs=[
            pl.BlockSpec((_BT, _H), lambda i: (i, 0)),
            pl.BlockSpec((1, _H), lambda i: (0, 0)),
            pl.BlockSpec((1, 1), lambda i: (0, 0)),
        ],
        out_specs=[
            pl.BlockSpec((1, 1, _BT), lambda i: (i, 0, 0)),
            pl.BlockSpec((1, 1), lambda i: (0, 0)),
        ],
        out_shape=[
            jax.ShapeDtypeStruct((_NB, 1, _BT), jnp.float32),
            jax.ShapeDtypeStruct((1, 1), jnp.float32),
        ],
    )(enc_flat, Wg, bg.reshape(1, 1))


def _select_body(s_ref, idx_ref):
    i32, f32 = jnp.int32, jnp.float32
    scores = s_ref[...]

    lo = jnp.zeros((_B, 1, 1), i32)
    hi = jnp.full((_B, 1, 1), 0x3F800000, i32)

    def bs_body(_, carry):
        lo, hi = carry
        mid = (lo + hi + 1) // 2
        midf = jax.lax.bitcast_convert_type(mid, f32)
        m = (scores >= midf).astype(i32)
        cnt = jnp.sum(jnp.sum(m, axis=2, keepdims=True), axis=1, keepdims=True)
        ok = cnt >= _K
        return jnp.where(ok, mid, lo), jnp.where(ok, hi, mid - 1)

    lo, hi = jax.lax.fori_loop(0, 31, bs_body, (lo, hi))
    thr = jax.lax.bitcast_convert_type(lo, f32)

    gt = scores > thr
    eq = scores == thr
    c_gt = jnp.sum(jnp.sum(gt.astype(i32), axis=2, keepdims=True),
                   axis=1, keepdims=True)
    need = _K - c_gt

    ci = jax.lax.broadcasted_iota(i32, (_B, 64, 128), 1)
    li = jax.lax.broadcasted_iota(i32, (_B, 64, 128), 2)
    tidx = ci * 128 + li

    lo2 = jnp.zeros((_B, 1, 1), i32)
    hi2 = jnp.full((_B, 1, 1), _T, i32)

    def bs2_body(_, carry):
        lo2, hi2 = carry
        mid = (lo2 + hi2) // 2
        g = jnp.sum(jnp.sum((eq & (tidx < mid)).astype(i32), axis=2,
                            keepdims=True), axis=1, keepdims=True)
        ok = g >= need
        return jnp.where(ok, lo2, mid + 1), jnp.where(ok, mid, hi2)

    lo2, _ = jax.lax.fori_loop(0, 14, bs2_body, (lo2, hi2))

    maskf = (gt | (eq & (tidx < lo2))).astype(f32)

    S = jnp.sum(maskf, axis=2)
    r64 = jax.lax.broadcasted_iota(i32, (64, 64), 0)
    c64 = jax.lax.broadcasted_iota(i32, (64, 64), 1)
    O = jax.lax.dot_general(S, (r64 < c64).astype(f32),
                            (((1,), (0,)), ((), ())), precision=_HIGH)
    r128 = jax.lax.broadcasted_iota(i32, (128, 128), 0)
    c128 = jax.lax.broadcasted_iota(i32, (128, 128), 1)
    p = jax.lax.dot_general(maskf, (r128 <= c128).astype(f32),
                            (((2,), (0,)), ((), ())), precision=_HIGH)

    jj = jax.lax.broadcasted_iota(i32, (_B, 64, 128), 2).astype(f32)
    O3 = O[:, :, None]
    S3 = S[:, :, None]
    c_onehot = ((O3 <= jj) & (jj < O3 + S3)).astype(f32)

    cif = jax.lax.broadcasted_iota(i32, (_B, 64, 128), 1).astype(f32)
    cvals = jnp.sum(c_onehot * cif, axis=1)
    O_sel = jnp.sum(c_onehot * O3, axis=1)
    jf = jax.lax.broadcasted_iota(i32, (_B, 128), 1).astype(f32)
    r = jf - O_sel

    p_sel = jax.lax.dot_general(c_onehot, p, (((1,), (1,)), ((0,), (0,))),
                                precision=_HIGH)
    lvals = jnp.sum((p_sel <= r[:, :, None]).astype(f32), axis=2)

    bi = jax.lax.broadcasted_iota(i32, (_B, 128), 0).astype(f32)
    idx_ref[...] = (cvals * 128.0 + lvals + bi * float(_T)).astype(i32)


def _select_call(gate3):
    return pl.pallas_call(
        _select_body,
        out_shape=jax.ShapeDtypeStruct((_B, _K), jnp.int32),
    )(gate3)


def _sc_gather_body(table_hbm, idx_hbm, out_hbm, idx_v, rows_v, sem):
    wid = jax.lax.axis_index("s") * 2 + jax.lax.axis_index("c")
    base = wid * _RPW
    pltpu.sync_copy(idx_hbm.at[pl.ds(base, _RPW)], idx_v)
    pltpu.async_copy(table_hbm.at[idx_v], rows_v, sem).wait()
    pltpu.sync_copy(rows_v, out_hbm.at[pl.ds(base, _RPW)])


_sc_gather = functools.partial(
    pl.kernel,
    mesh=plsc.VectorSubcoreMesh(core_axis_name="c", subcore_axis_name="s"),
    out_type=jax.ShapeDtypeStruct((_B * _K, _H), jnp.float32),
    scratch_types=[
        pltpu.VMEM((_RPW,), jnp.int32),
        pltpu.VMEM((_RPW, _H), jnp.float32),
        pltpu.SemaphoreType.DMA,
    ],
)(_sc_gather_body)


def _qk_body(qh_ref, wq_ref, bq_ref, wk_ref, bk_ref, u_ref, s0_ref):
    qh = qh_ref[...]
    q = jax.lax.dot_general(qh, wq_ref[...], (((1,), (1,)), ((), ())),
                            precision=_HIGH) + bq_ref[...]
    u_ref[...] = jax.lax.dot_general(q, wk_ref[...], (((1,), (0,)), ((), ())),
                                     precision=_HIGH)
    s0_ref[...] = jnp.sum(q * bk_ref[...], axis=1, keepdims=True)


def _qk_call(query_hidden, Wq, bq, Wk, bk):
    return pl.pallas_call(
        _qk_body,
        out_shape=[
            jax.ShapeDtypeStruct((_B, _H), jnp.float32),
            jax.ShapeDtypeStruct((_B, 1), jnp.float32),
        ],
    )(query_hidden, Wq, bq.reshape(1, _H), Wk, bk.reshape(1, _H))


def _attn_body(g_ref, qh_ref, u_ref, s0_ref, wo_ref,
               bo_ref, gsum_ref, logits_ref, wr_ref):
    g = g_ref[...]
    qh = qh_ref[...]
    u = u_ref[...]
    s0 = s0_ref[...]

    inv = 1.0 / (_H ** 0.5)
    sc = jax.lax.dot_general(g, u, (((2,), (1,)), ((0,), (0,))),
                             precision=_HIGH) * inv
    s0p = s0 * inv
    m = jnp.maximum(jnp.max(sc, axis=1, keepdims=True), s0p)
    e = jnp.exp(sc - m)
    e0 = jnp.exp(s0p - m)
    denom = jnp.sum(e, axis=1, keepdims=True) + float(_M - _K) * e0
    attn = e / denom

    retr = jax.lax.dot_general(attn, g, (((1,), (1,)), ((0,), (0,))),
                               precision=_HIGH)
    out = retr + qh
    logits_ref[...] = jax.lax.dot_general(
        out, wo_ref[...], (((1,), (1,)), ((), ())),
        precision=_HIGH) + bo_ref[...]
    wr_ref[...] = gsum_ref[...] * (1.0 / float(_B * _T))


def _attn_call(gathered3, query_hidden, u, s0, Wo, bo, gsum):
    return pl.pallas_call(
        _attn_body,
        out_shape=[
            jax.ShapeDtypeStruct((_B, _V), jnp.float32),
            jax.ShapeDtypeStruct((1, 1), jnp.float32),
        ],
    )(gathered3, query_hidden, u, s0, Wo, bo.reshape(1, _V), gsum)


def kernel(enc_hidden, query_hidden, Wg, bg, Wq, bq, Wk, bk, Wo, bo):
    enc_flat = enc_hidden.reshape(_B * _T, _H)
    gate3, gsum = _gate_call(enc_flat, Wg, bg)
    gate_scores = gate3.reshape(_B, _T)

    idx = _select_call(gate_scores.reshape(_B, 64, 128))
    gathered = _sc_gather(enc_flat, idx.reshape(_B * _K))
    u, s0 = _qk_call(query_hidden, Wq, bq, Wk, bk)

    logits, wr = _attn_call(gathered.reshape(_B, _K, _H), query_hidden,
                            u, s0, Wo, bo, gsum)
    return (logits, gate_scores, wr.reshape(()))


# speedup vs baseline: 1.0147x; 1.0147x over previous
import functools

import jax
import jax.numpy as jnp
from jax.experimental import pallas as pl
from jax.experimental.pallas import tpu as pltpu
from jax.experimental.pallas import tpu_sc as plsc

_H = 1024
_M = 1024
_K = 128
_B = 4
_T = 8192
_V = 64

_BT = 4096
_NB = (_B * _T) // _BT
_NW = 32
_RPW = (_B * _K) // _NW

_HIGH = jax.lax.Precision.HIGHEST


def _gate_body(x_ref, wg_ref, bg_ref, gate_ref, acc_ref):
    i = pl.program_id(0)
    x = x_ref[...]
    w = wg_ref[...]
    y = jnp.sum(x * w, axis=1)
    sig = jax.nn.sigmoid(y + bg_ref[0, 0])
    gate_ref[...] = sig.reshape(1, 1, _BT)

    @pl.when(i == 0)
    def _():
        acc_ref[...] = jnp.zeros_like(acc_ref)

    acc_ref[...] += jnp.sum(sig).reshape(1, 1)


def _gate_call(enc_flat, Wg, bg):
    return pl.pallas_call(
        _gate_body,
        grid=(_NB,),
        in_specs=[
            pl.BlockSpec((_BT, _H), lambda i: (i, 0)),
            pl.BlockSpec((1, _H), lambda i: (0, 0)),
            pl.BlockSpec((1, 1), lambda i: (0, 0)),
        ],
        out_specs=[
            pl.BlockSpec((1, 1, _BT), lambda i: (i, 0, 0)),
            pl.BlockSpec((1, 1), lambda i: (0, 0)),
        ],
        out_shape=[
            jax.ShapeDtypeStruct((_NB, 1, _BT), jnp.float32),
            jax.ShapeDtypeStruct((1, 1), jnp.float32),
        ],
    )(enc_flat, Wg, bg.reshape(1, 1))


def _select_body(s_ref, idx_ref):
    i32, f32 = jnp.int32, jnp.float32
    scores = s_ref[...]

    lo = jnp.zeros((_B, 1, 1), i32)
    hi = jnp.full((_B, 1, 1), 0x3F800000, i32)

    def bs_body(_, carry):
        lo, hi = carry
        mid = (lo + hi + 1) // 2
        midf = jax.lax.bitcast_convert_type(mid, f32)
        m = (scores >= midf).astype(i32)
        cnt = jnp.sum(jnp.sum(m, axis=2, keepdims=True), axis=1, keepdims=True)
        ok = cnt >= _K
        return jnp.where(ok, mid, lo), jnp.where(ok, hi, mid - 1)

    lo, hi = jax.lax.fori_loop(0, 31, bs_body, (lo, hi))
    thr = jax.lax.bitcast_convert_type(lo, f32)

    gt = scores > thr
    eq = scores == thr
    c_gt = jnp.sum(jnp.sum(gt.astype(i32), axis=2, keepdims=True),
                   axis=1, keepdims=True)
    need = _K - c_gt

    ci = jax.lax.broadcasted_iota(i32, (_B, 64, 128), 1)
    li = jax.lax.broadcasted_iota(i32, (_B, 64, 128), 2)
    tidx = ci * 128 + li

    n_eq = jnp.sum(jnp.sum(eq.astype(i32), axis=2, keepdims=True),
                   axis=1, keepdims=True)

    def _tie_search():
        lo2 = jnp.zeros((_B, 1, 1), i32)
        hi2 = jnp.full((_B, 1, 1), _T, i32)

        def bs2_body(_, carry):
            lo2, hi2 = carry
            mid = (lo2 + hi2) // 2
            g = jnp.sum(jnp.sum((eq & (tidx < mid)).astype(i32), axis=2,
                                keepdims=True), axis=1, keepdims=True)
            ok = g >= need
            return jnp.where(ok, lo2, mid + 1), jnp.where(ok, mid, hi2)

        lo2, _ = jax.lax.fori_loop(0, 14, bs2_body, (lo2, hi2))
        return lo2

    cutoff = jax.lax.cond(jnp.all(n_eq == need),
                          lambda: jnp.full((_B, 1, 1), _T, i32), _tie_search)

    maskf = (gt | (eq & (tidx < cutoff))).astype(f32)

    S = jnp.sum(maskf, axis=2)
    r64 = jax.lax.broadcasted_iota(i32, (64, 64), 0)
    c64 = jax.lax.broadcasted_iota(i32, (64, 64), 1)
    O = jax.lax.dot_general(S, (r64 < c64).astype(f32),
                            (((1,), (0,)), ((), ())), precision=_HIGH)
    r128 = jax.lax.broadcasted_iota(i32, (128, 128), 0)
    c128 = jax.lax.broadcasted_iota(i32, (128, 128), 1)
    p = jax.lax.dot_general(maskf, (r128 <= c128).astype(f32),
                            (((2,), (0,)), ((), ())), precision=_HIGH)

    jj = jax.lax.broadcasted_iota(i32, (_B, 64, 128), 2).astype(f32)
    O3 = O[:, :, None]
    S3 = S[:, :, None]
    c_onehot = ((O3 <= jj) & (jj < O3 + S3)).astype(f32)

    cif = jax.lax.broadcasted_iota(i32, (_B, 64, 128), 1).astype(f32)
    cvals = jnp.sum(c_onehot * cif, axis=1)
    O_sel = jnp.sum(c_onehot * O3, axis=1)
    jf = jax.lax.broadcasted_iota(i32, (_B, 128), 1).astype(f32)
    r = jf - O_sel

    p_sel = jax.lax.dot_general(c_onehot, p, (((1,), (1,)), ((0,), (0,))),
                                precision=_HIGH)
    lvals = jnp.sum((p_sel <= r[:, :, None]).astype(f32), axis=2)

    bi = jax.lax.broadcasted_iota(i32, (_B, 128), 0).astype(f32)
    idx_ref[...] = (cvals * 128.0 + lvals + bi * float(_T)).astype(i32)


def _select_call(gate3):
    return pl.pallas_call(
        _select_body,
        out_shape=jax.ShapeDtypeStruct((_B, _K), jnp.int32),
    )(gate3)


def _sc_gather_body(table_hbm, idx_hbm, out_hbm, idx_v, rows_v, sem):
    wid = jax.lax.axis_index("s") * 2 + jax.lax.axis_index("c")
    base = wid * _RPW
    pltpu.sync_copy(idx_hbm.at[pl.ds(base, _RPW)], idx_v)
    pltpu.async_copy(table_hbm.at[idx_v], rows_v, sem).wait()
    pltpu.sync_copy(rows_v, out_hbm.at[pl.ds(base, _RPW)])


_sc_gather = functools.partial(
    pl.kernel,
    mesh=plsc.VectorSubcoreMesh(core_axis_name="c", subcore_axis_name="s"),
    out_type=jax.ShapeDtypeStruct((_B * _K, _H), jnp.float32),
    scratch_types=[
        pltpu.VMEM((_RPW,), jnp.int32),
        pltpu.VMEM((_RPW, _H), jnp.float32),
        pltpu.SemaphoreType.DMA,
    ],
)(_sc_gather_body)


def _qk_body(qh_ref, wq_ref, bq_ref, wk_ref, bk_ref, u_ref, s0_ref):
    qh = qh_ref[...]
    q = jax.lax.dot_general(qh, wq_ref[...], (((1,), (1,)), ((), ())),
                            precision=_HIGH) + bq_ref[...]
    u_ref[...] = jax.lax.dot_general(q, wk_ref[...], (((1,), (0,)), ((), ())),
                                     precision=_HIGH)
    s0_ref[...] = jnp.sum(q * bk_ref[...], axis=1, keepdims=True)


def _qk_call(query_hidden, Wq, bq, Wk, bk):
    return pl.pallas_call(
        _qk_body,
        out_shape=[
            jax.ShapeDtypeStruct((_B, _H), jnp.float32),
            jax.ShapeDtypeStruct((_B, 1), jnp.float32),
        ],
    )(query_hidden, Wq, bq.reshape(1, _H), Wk, bk.reshape(1, _H))


def _attn_body(g_ref, qh_ref, u_ref, s0_ref, wo_ref,
               bo_ref, gsum_ref, logits_ref, wr_ref):
    g = g_ref[...]
    qh = qh_ref[...]
    u = u_ref[...]
    s0 = s0_ref[...]

    inv = 1.0 / (_H ** 0.5)
    sc = jax.lax.dot_general(g, u, (((2,), (1,)), ((0,), (0,))),
                             precision=_HIGH) * inv
    s0p = s0 * inv
    m = jnp.maximum(jnp.max(sc, axis=1, keepdims=True), s0p)
    e = jnp.exp(sc - m)
    e0 = jnp.exp(s0p - m)
    denom = jnp.sum(e, axis=1, keepdims=True) + float(_M - _K) * e0
    attn = e / denom

    retr = jax.lax.dot_general(attn, g, (((1,), (1,)), ((0,), (0,))),
                               precision=_HIGH)
    out = retr + qh
    logits_ref[...] = jax.lax.dot_general(
        out, wo_ref[...], (((1,), (1,)), ((), ())),
        precision=_HIGH) + bo_ref[...]
    wr_ref[...] = gsum_ref[...] * (1.0 / float(_B * _T))


def _attn_call(gathered3, query_hidden, u, s0, Wo, bo, gsum):
    return pl.pallas_call(
        _attn_body,
        out_shape=[
            jax.ShapeDtypeStruct((_B, _V), jnp.float32),
            jax.ShapeDtypeStruct((1, 1), jnp.float32),
        ],
    )(gathered3, query_hidden, u, s0, Wo, bo.reshape(1, _V), gsum)


def kernel(enc_hidden, query_hidden, Wg, bg, Wq, bq, Wk, bk, Wo, bo):
    enc_flat = enc_hidden.reshape(_B * _T, _H)
    gate3, gsum = _gate_call(enc_flat, Wg, bg)
    gate_scores = gate3.reshape(_B, _T)

    idx = _select_call(gate_scores.reshape(_B, 64, 128))
    gathered = _sc_gather(enc_flat, idx.reshape(_B * _K))
    u, s0 = _qk_call(query_hidden, Wq, bq, Wk, bk)

    logits, wr = _attn_call(gathered.reshape(_B, _K, _H), query_hidden,
                            u, s0, Wo, bo, gsum)
    return (logits, gate_scores, wr.reshape(()))


# speedup vs baseline: 1.1883x; 1.1712x over previous
import functools

import jax
import jax.numpy as jnp
from jax.experimental import pallas as pl
from jax.experimental.pallas import tpu as pltpu
from jax.experimental.pallas import tpu_sc as plsc

_H = 1024
_M = 1024
_K = 128
_B = 4
_T = 8192
_V = 64

_BT = 4096
_NB = (_B * _T) // _BT
_NW = 32
_RPW = (_B * _K) // _NW

_HIGH = jax.lax.Precision.HIGHEST


def _gate_body(x_ref, wg_ref, bg_ref, gate_ref, acc_ref):
    i = pl.program_id(0)
    x = x_ref[...]
    w = wg_ref[...]
    y = jnp.sum(x * w, axis=1)
    sig = jax.nn.sigmoid(y + bg_ref[0, 0])
    gate_ref[...] = sig.reshape(1, 1, _BT)

    @pl.when(i == 0)
    def _():
        acc_ref[...] = jnp.zeros_like(acc_ref)

    acc_ref[...] += jnp.sum(sig).reshape(1, 1)


def _gate_call(enc_flat, Wg, bg):
    return pl.pallas_call(
        _gate_body,
        grid=(_NB,),
        in_specs=[
            pl.BlockSpec((_BT, _H), lambda i: (i, 0)),
            pl.BlockSpec((1, _H), lambda i: (0, 0)),
            pl.BlockSpec((1, 1), lambda i: (0, 0)),
        ],
        out_specs=[
            pl.BlockSpec((1, 1, _BT), lambda i: (i, 0, 0)),
            pl.BlockSpec((1, 1), lambda i: (0, 0)),
        ],
        out_shape=[
            jax.ShapeDtypeStruct((_NB, 1, _BT), jnp.float32),
            jax.ShapeDtypeStruct((1, 1), jnp.float32),
        ],
    )(enc_flat, Wg, bg.reshape(1, 1))


def _select_body(s_ref, idx_ref):
    i32, f32 = jnp.int32, jnp.float32
    scores = s_ref[...]

    lo = jnp.zeros((_B, 1, 1), i32)
    hi = jnp.full((_B, 1, 1), 0x3F800000, i32)

    def bs_body(_, carry):
        lo, hi = carry
        mid = (lo + hi + 1) // 2
        midf = jax.lax.bitcast_convert_type(mid, f32)
        m = (scores >= midf).astype(i32)
        cnt = jnp.sum(jnp.sum(m, axis=2, keepdims=True), axis=1, keepdims=True)
        ok = cnt >= _K
        return jnp.where(ok, mid, lo), jnp.where(ok, hi, mid - 1)

    lo, hi = jax.lax.fori_loop(0, 31, bs_body, (lo, hi))
    thr = jax.lax.bitcast_convert_type(lo, f32)

    gt = scores > thr
    eq = scores == thr
    c_gt = jnp.sum(jnp.sum(gt.astype(i32), axis=2, keepdims=True),
                   axis=1, keepdims=True)
    need = _K - c_gt

    ci = jax.lax.broadcasted_iota(i32, (_B, 64, 128), 1)
    li = jax.lax.broadcasted_iota(i32, (_B, 64, 128), 2)
    tidx = ci * 128 + li

    n_eq = jnp.sum(jnp.sum(eq.astype(i32), axis=2, keepdims=True),
                   axis=1, keepdims=True)

    def _tie_search():
        lo2 = jnp.zeros((_B, 1, 1), i32)
        hi2 = jnp.full((_B, 1, 1), _T, i32)

        def bs2_body(_, carry):
            lo2, hi2 = carry
            mid = (lo2 + hi2) // 2
            g = jnp.sum(jnp.sum((eq & (tidx < mid)).astype(i32), axis=2,
                                keepdims=True), axis=1, keepdims=True)
            ok = g >= need
            return jnp.where(ok, lo2, mid + 1), jnp.where(ok, mid, hi2)

        lo2, _ = jax.lax.fori_loop(0, 14, bs2_body, (lo2, hi2))
        return lo2

    cutoff = jax.lax.cond(jnp.all(n_eq == need),
                          lambda: jnp.full((_B, 1, 1), _T, i32), _tie_search)

    maskf = (gt | (eq & (tidx < cutoff))).astype(f32)

    S = jnp.sum(maskf, axis=2)
    r64 = jax.lax.broadcasted_iota(i32, (64, 64), 0)
    c64 = jax.lax.broadcasted_iota(i32, (64, 64), 1)
    O = jax.lax.dot_general(S, (r64 < c64).astype(f32),
                            (((1,), (0,)), ((), ())), precision=_HIGH)
    r128 = jax.lax.broadcasted_iota(i32, (128, 128), 0)
    c128 = jax.lax.broadcasted_iota(i32, (128, 128), 1)
    p = jax.lax.dot_general(maskf, (r128 <= c128).astype(f32),
                            (((2,), (0,)), ((), ())), precision=_HIGH)

    jj = jax.lax.broadcasted_iota(i32, (_B, 64, 128), 2).astype(f32)
    O3 = O[:, :, None]
    S3 = S[:, :, None]
    c_onehot = ((O3 <= jj) & (jj < O3 + S3)).astype(f32)

    cif = jax.lax.broadcasted_iota(i32, (_B, 64, 128), 1).astype(f32)
    cvals = jnp.sum(c_onehot * cif, axis=1)
    O_sel = jnp.sum(c_onehot * O3, axis=1)
    jf = jax.lax.broadcasted_iota(i32, (_B, 128), 1).astype(f32)
    r = jf - O_sel

    p_sel = jax.lax.dot_general(c_onehot, p, (((1,), (1,)), ((0,), (0,))),
                                precision=_HIGH)
    lvals = jnp.sum((p_sel <= r[:, :, None]).astype(f32), axis=2)

    bi = jax.lax.broadcasted_iota(i32, (_B, 128), 0).astype(f32)
    idx_ref[...] = (cvals * 128.0 + lvals + bi * float(_T)).astype(i32)


def _select_call(gate3):
    return pl.pallas_call(
        _select_body,
        out_shape=jax.ShapeDtypeStruct((_B, _K), jnp.int32),
    )(gate3)


def _tail_body(idx_ref, enc_ref, qh_ref, wq_ref, bq_ref, wk_ref, bk_ref,
               wo_ref, bo_ref, gsum_ref, logits_ref, wr_ref, rows_ref, sem):
    def issue(j, carry):
        r = idx_ref[j]
        pltpu.make_async_copy(enc_ref.at[pl.ds(r, 1), :],
                              rows_ref.at[pl.ds(j, 1), :], sem).start()
        return carry

    jax.lax.fori_loop(0, _B * _K, issue, 0)

    qh = qh_ref[...]
    q = jax.lax.dot_general(qh, wq_ref[...], (((1,), (1,)), ((), ())),
                            precision=_HIGH) + bq_ref[...]
    u = jax.lax.dot_general(q, wk_ref[...], (((1,), (0,)), ((), ())),
                            precision=_HIGH)
    s0 = jnp.sum(q * bk_ref[...], axis=1, keepdims=True)

    def drain(j, carry):
        pltpu.make_async_copy(enc_ref.at[pl.ds(0, 1), :],
                              rows_ref.at[pl.ds(j, 1), :], sem).wait()
        return carry

    jax.lax.fori_loop(0, _B * _K, drain, 0)

    g = rows_ref[...].reshape(_B, _K, _H)
    inv = 1.0 / (_H ** 0.5)
    sc = jax.lax.dot_general(g, u, (((2,), (1,)), ((0,), (0,))),
                             precision=_HIGH) * inv
    s0p = s0 * inv
    m = jnp.maximum(jnp.max(sc, axis=1, keepdims=True), s0p)
    e = jnp.exp(sc - m)
    e0 = jnp.exp(s0p - m)
    denom = jnp.sum(e, axis=1, keepdims=True) + float(_M - _K) * e0
    attn = e / denom

    retr = jax.lax.dot_general(attn, g, (((1,), (1,)), ((0,), (0,))),
                               precision=_HIGH)
    out = retr + qh
    logits_ref[...] = jax.lax.dot_general(
        out, wo_ref[...], (((1,), (1,)), ((), ())),
        precision=_HIGH) + bo_ref[...]
    wr_ref[...] = gsum_ref[...] * (1.0 / float(_B * _T))


def _tail_call(idx_flat, enc_flat, query_hidden, Wq, bq, Wk, bk, Wo, bo, gsum):
    return pl.pallas_call(
        _tail_body,
        in_specs=[
            pl.BlockSpec(memory_space=pltpu.SMEM),
            pl.BlockSpec(memory_space=pl.ANY),
            pl.BlockSpec(memory_space=pltpu.VMEM),
            pl.BlockSpec(memory_space=pltpu.VMEM),
            pl.BlockSpec(memory_space=pltpu.VMEM),
            pl.BlockSpec(memory_space=pltpu.VMEM),
            pl.BlockSpec(memory_space=pltpu.VMEM),
            pl.BlockSpec(memory_space=pltpu.VMEM),
            pl.BlockSpec(memory_space=pltpu.VMEM),
            pl.BlockSpec(memory_space=pltpu.VMEM),
        ],
        scratch_shapes=[
            pltpu.VMEM((_B * _K, _H), jnp.float32),
            pltpu.SemaphoreType.DMA,
        ],
        out_shape=[
            jax.ShapeDtypeStruct((_B, _V), jnp.float32),
            jax.ShapeDtypeStruct((1, 1), jnp.float32),
        ],
    )(idx_flat, enc_flat, query_hidden, Wq, bq.reshape(1, _H), Wk,
      bk.reshape(1, _H), Wo, bo.reshape(1, _V), gsum)


def kernel(enc_hidden, query_hidden, Wg, bg, Wq, bq, Wk, bk, Wo, bo):
    enc_flat = enc_hidden.reshape(_B * _T, _H)
    gate3, gsum = _gate_call(enc_flat, Wg, bg)
    gate_scores = gate3.reshape(_B, _T)

    idx = _select_call(gate_scores.reshape(_B, 64, 128))
    logits, wr = _tail_call(idx.reshape(_B * _K), enc_flat, query_hidden,
                            Wq, bq, Wk, bk, Wo, bo, gsum)
    return (logits, gate_scores, wr.reshape(()))
